# baseline (speedup 1.0000x reference)
import functools

import jax
import jax.numpy as jnp
from jax import lax
from jax.experimental import pallas as pl
from jax.experimental.pallas import tpu as pltpu
from jax.experimental.pallas import tpu_sc as plsc

N = 100000
D = 128
NG2 = D // 32
NF = 7

NC = 2
NS = 16
NW = NC * NS

C = 80
NCHUNK = N // C

R_FC, R_DEG, R_CH, R_NH, R_AR, R_HY = 0, 22, 39, 53, 68, 75
T_C1, T_C2, T_C3 = 124, 278, 516
V = 726

_mesh = plsc.VectorSubcoreMesh(
    core_axis_name="c", subcore_axis_name="s", num_cores=NC, num_subcores=NS
)


@functools.partial(
    pl.kernel,
    out_type=jax.ShapeDtypeStruct((N, D), jnp.float32),
    mesh=_mesh,
    scratch_types=[
        pltpu.VMEM((V, D // 2), jnp.int32),
        pltpu.VMEM((96, D), jnp.float32),
        [pltpu.VMEM((NF * C,), jnp.int32) for _ in range(2)],
        [pltpu.VMEM((C, D), jnp.float32) for _ in range(2)],
        [pltpu.SemaphoreType.DMA for _ in range(2)],
        [pltpu.SemaphoreType.DMA for _ in range(2)],
    ],
)
def _embed_sum(idx_hbm, wa_hbm, wr_hbm, out_hbm,
               tab_v, raw_v, idx_vs, acc_vs, sem_idx, sem_out):
    wid = lax.axis_index("s") * NC + lax.axis_index("c")

    nmine = (NCHUNK - 1 - wid) // NW + 1

    def issue_idx(t, b):
        base = (wid + t * NW) * (NF * C)
        pltpu.async_copy(idx_hbm.at[pl.ds(base, NF * C)],
                         idx_vs[b], sem_idx[b])

    def wait_idx(b):
        pltpu.make_async_copy(idx_hbm.at[pl.ds(0, NF * C)],
                              idx_vs[b], sem_idx[b]).wait()

    issue_idx(0, 0)

    pltpu.sync_copy(wr_hbm, raw_v.at[pl.ds(0, 89), :])

    M_HI = jnp.int32(-65536)
    HALF = jnp.int32(0x8000)

    def pack_row(dst_row, vals):
        for c in range(NG2):
            ai = lax.bitcast_convert_type(vals[2 * c], jnp.int32)
            bi = lax.bitcast_convert_type(vals[2 * c + 1], jnp.int32)
            lo = lax.shift_right_logical(ai + HALF, 16)
            tab_v[dst_row, pl.ds(c * 16, 16)] = lo | (bi & M_HI)

    def build_pair(i, _, *, nb, ra, rb, tbase):
        avs = [raw_v[ra + i, pl.ds(g * 16, 16)] for g in range(8)]
        for j in range(nb):
            pack_row(tbase + i * nb + j,
                     [avs[g] + raw_v[rb + j, pl.ds(g * 16, 16)]
                      for g in range(8)])
        return 0

    lax.fori_loop(0, 22, functools.partial(
        build_pair, nb=7, ra=R_FC, rb=R_AR, tbase=T_C1), 0)
    lax.fori_loop(0, 17, functools.partial(
        build_pair, nb=14, ra=R_DEG, rb=R_CH, tbase=T_C2), 0)
    lax.fori_loop(0, 15, functools.partial(
        build_pair, nb=14, ra=R_NH, rb=R_HY, tbase=T_C3), 0)

    def build_at(i, _, *, tb):
        pack_row(tb + i, [raw_v[i, pl.ds(g * 16, 16)] for g in range(8)])
        return 0

    pltpu.sync_copy(wa_hbm.at[pl.ds(0, 96), :], raw_v)
    lax.fori_loop(0, 96, functools.partial(build_at, tb=0), 0)
    pltpu.sync_copy(wa_hbm.at[pl.ds(96, 28), :], raw_v.at[pl.ds(0, 28), :])
    lax.fori_loop(0, 28, functools.partial(build_at, tb=96), 0)

    def do_chunk(t, b):
        dix = idx_vs[b]
        acc_v = acc_vs[b]
        base = (wid + t * NW) * C

        def blk_body(ib, _):
            i0 = ib * 16
            def dxf(f):
                return dix[pl.ds(f * C + i0, 16)]

            v_at = dxf(0)
            v_c1 = dxf(1) * 7 + dxf(5) + T_C1
            v_c2 = dxf(2) * 14 + dxf(3) + T_C2
            v_c3 = dxf(4) * 14 + dxf(6) + T_C3
            def lo_f32(w):
                return lax.bitcast_convert_type(
                    lax.shift_left(w, 16), jnp.float32)

            def hi_f32(w):
                return lax.bitcast_convert_type(w, jnp.float32)

            def get_rs(k):
                return [(v_at[k + q], v_c1[k + q], v_c2[k + q],
                         v_c3[k + q]) for q in range(4)]

            def emit_loads(rs, g):
                sl = pl.ds(g * 16, 16)
                return [tab_v[rs[q][t_], sl]
                        for q in range(4) for t_ in range(4)]

            regions = [(k, g) for k in range(0, 16, 4)
                       for g in range(NG2)]
            rs_cache = {0: get_rs(0)}
            cur = emit_loads(rs_cache[0], 0)
            for i, (k, g) in enumerate(regions):
                if g == 2 and k + 4 < 16:
                    rs_cache[k + 4] = get_rs(k + 4)
                nxt = None
                if i + 1 < len(regions):
                    k2, g2 = regions[i + 1]
                    nxt = emit_loads(rs_cache[k2], g2)
                for q in range(4):
                    w0, w1, w2, w3 = cur[q * 4:(q + 1) * 4]
                    e = ((lo_f32(w0) + lo_f32(w1))
                         + (lo_f32(w2) + lo_f32(w3)))
                    o = ((hi_f32(w0) + hi_f32(w1))
                         + (hi_f32(w2) + hi_f32(w3)))
                    acc_v[i0 + k + q, pl.ds(g * 32, 16)] = e
                    acc_v[i0 + k + q, pl.ds(g * 32 + 16, 16)] = o
                cur = nxt
            return 0

        lax.fori_loop(0, C // 16, blk_body, 0)
        pltpu.async_copy(acc_v, out_hbm.at[pl.ds(base, C), :], sem_out[b])

    def wait_out(b):
        pltpu.make_async_copy(acc_vs[b], out_hbm.at[pl.ds(0, C), :],
                              sem_out[b]).wait()

    def pair_body(p, _):
        for sub in range(2):
            t = p * 2 + sub
            b = sub

            @pl.when(t < nmine)
            def _():
                @pl.when(t + 1 < nmine)
                def _():
                    issue_idx(t + 1, 1 - b)
                wait_idx(b)

                @pl.when(t >= 2)
                def _():
                    wait_out(b)
                do_chunk(t, b)
        return 0

    lax.fori_loop(0, (nmine + 1) // 2, pair_body, 0)
    wait_out(0)
    wait_out(1)


def kernel(atomic_num, formal_charge, degree, chiral_tag, total_numHs,
           is_aromatic, hybridization, W_atomic_num, W_formal_charge,
           W_degree, W_chiral_tag, W_total_numHs, W_is_aromatic,
           W_hybridization):
    idx = jnp.stack([atomic_num, formal_charge, degree, chiral_tag,
                     total_numHs, is_aromatic, hybridization])
    idx = idx.astype(jnp.int32).reshape(NF, NCHUNK, C)
    idx = idx.transpose(1, 0, 2).reshape(-1)
    w_rest = jnp.concatenate([W_formal_charge, W_degree, W_chiral_tag,
                              W_total_numHs, W_is_aromatic,
                              W_hybridization], axis=0)
    return _embed_sum(idx, W_atomic_num, w_rest)


# speedup vs baseline: 1.2236x; 1.2236x over previous
import functools

import jax
import jax.numpy as jnp
from jax import lax
from jax.experimental import pallas as pl
from jax.experimental.pallas import tpu as pltpu
from jax.experimental.pallas import tpu_sc as plsc

N = 100000
D = 128
NG2 = D // 32
NF = 7

NC = 2
NS = 16
NW = NC * NS

C = 80
NCHUNK = N // C

R_FC, R_DEG, R_CH, R_NH, R_AR, R_HY = 0, 24, 48, 64, 80, 88
T_C1, T_C2, T_C3 = 124, 278, 516
V = 726

_mesh = plsc.VectorSubcoreMesh(
    core_axis_name="c", subcore_axis_name="s", num_cores=NC, num_subcores=NS
)


@functools.partial(
    pl.kernel,
    out_type=jax.ShapeDtypeStruct((N, D), jnp.float32),
    mesh=_mesh,
    scratch_types=[
        pltpu.VMEM((V, D // 2), jnp.int32),
        pltpu.VMEM((104, D), jnp.float32),
        [[pltpu.VMEM((C,), jnp.int32) for _ in range(NF)] for _ in range(2)],
        [pltpu.VMEM((C, D), jnp.float32) for _ in range(2)],
        [pltpu.SemaphoreType.DMA for _ in range(2)],
        [pltpu.SemaphoreType.DMA for _ in range(2)],
    ],
)
def _embed_sum(i0_hbm, i1_hbm, i2_hbm, i3_hbm, i4_hbm, i5_hbm, i6_hbm,
               wa_hbm, wfc_hbm, wdeg_hbm, wch_hbm, wnh_hbm, war_hbm,
               why_hbm, out_hbm,
               tab_v, raw_v, idx_vs, acc_vs, sem_idx, sem_out):
    wid = lax.axis_index("s") * NC + lax.axis_index("c")
    idx_hbms = (i0_hbm, i1_hbm, i2_hbm, i3_hbm, i4_hbm, i5_hbm, i6_hbm)

    nmine = (NCHUNK - 1 - wid) // NW + 1

    def issue_idx(t, b):
        base = (wid + t * NW) * C
        for f in range(NF):
            pltpu.async_copy(idx_hbms[f].at[pl.ds(base, C)],
                             idx_vs[b][f], sem_idx[b])

    def wait_idx(b):
        for f in range(NF):
            pltpu.make_async_copy(idx_hbms[f].at[pl.ds(0, C)],
                                  idx_vs[b][f], sem_idx[b]).wait()

    issue_idx(0, 0)

    w_srcs = (wfc_hbm, wdeg_hbm, wch_hbm, wnh_hbm, war_hbm, why_hbm)
    w_offs = (R_FC, R_DEG, R_CH, R_NH, R_AR, R_HY)
    w_sizes = (22, 17, 14, 15, 7, 14)
    for s, o, n in zip(w_srcs, w_offs, w_sizes):
        pltpu.async_copy(s, raw_v.at[pl.ds(o, n), :], sem_out[0])
    for s, o, n in zip(w_srcs, w_offs, w_sizes):
        pltpu.make_async_copy(s, raw_v.at[pl.ds(o, n), :],
                              sem_out[0]).wait()

    M_HI = jnp.int32(-65536)
    HALF = jnp.int32(0x8000)

    def pack_row(dst_row, vals):
        for c in range(NG2):
            ai = lax.bitcast_convert_type(vals[2 * c], jnp.int32)
            bi = lax.bitcast_convert_type(vals[2 * c + 1], jnp.int32)
            lo = lax.shift_right_logical(ai + HALF, 16)
            tab_v[dst_row, pl.ds(c * 16, 16)] = lo | (bi & M_HI)

    def build_pair(i, _, *, nb, ra, rb, tbase):
        avs = [raw_v[ra + i, pl.ds(g * 16, 16)] for g in range(8)]
        for j in range(nb):
            pack_row(tbase + i * nb + j,
                     [avs[g] + raw_v[rb + j, pl.ds(g * 16, 16)]
                      for g in range(8)])
        return 0

    lax.fori_loop(0, 22, functools.partial(
        build_pair, nb=7, ra=R_FC, rb=R_AR, tbase=T_C1), 0)
    lax.fori_loop(0, 17, functools.partial(
        build_pair, nb=14, ra=R_DEG, rb=R_CH, tbase=T_C2), 0)
    lax.fori_loop(0, 15, functools.partial(
        build_pair, nb=14, ra=R_NH, rb=R_HY, tbase=T_C3), 0)

    def build_at(i, _, *, tb):
        pack_row(tb + i, [raw_v[i, pl.ds(g * 16, 16)] for g in range(8)])
        return 0

    pltpu.sync_copy(wa_hbm.at[pl.ds(0, 96), :], raw_v.at[pl.ds(0, 96), :])
    lax.fori_loop(0, 96, functools.partial(build_at, tb=0), 0)
    pltpu.sync_copy(wa_hbm.at[pl.ds(96, 28), :], raw_v.at[pl.ds(0, 28), :])
    lax.fori_loop(0, 28, functools.partial(build_at, tb=96), 0)

    def do_chunk(t, b):
        dix = idx_vs[b]
        acc_v = acc_vs[b]
        base = (wid + t * NW) * C

        def blk_body(ib, _):
            i0 = ib * 16
            s16 = pl.ds(i0, 16)
            v_at = dix[0][s16]
            v_c1 = dix[1][s16] * 7 + dix[5][s16] + T_C1
            v_c2 = dix[2][s16] * 14 + dix[3][s16] + T_C2
            v_c3 = dix[4][s16] * 14 + dix[6][s16] + T_C3
            def lo_f32(w):
                return lax.bitcast_convert_type(
                    lax.shift_left(w, 16), jnp.float32)

            def hi_f32(w):
                return lax.bitcast_convert_type(w, jnp.float32)

            def get_rs(k):
                return [(v_at[k + q], v_c1[k + q], v_c2[k + q],
                         v_c3[k + q]) for q in range(4)]

            def emit_loads(rs, g):
                sl = pl.ds(g * 16, 16)
                return [tab_v[rs[q][t_], sl]
                        for q in range(4) for t_ in range(4)]

            regions = [(k, g) for k in range(0, 16, 4)
                       for g in range(NG2)]
            rs_cache = {0: get_rs(0)}
            cur = emit_loads(rs_cache[0], 0)
            for i, (k, g) in enumerate(regions):
                if g == 2 and k + 4 < 16:
                    rs_cache[k + 4] = get_rs(k + 4)
                nxt = None
                if i + 1 < len(regions):
                    k2, g2 = regions[i + 1]
                    nxt = emit_loads(rs_cache[k2], g2)
                for q in range(4):
                    w0, w1, w2, w3 = cur[q * 4:(q + 1) * 4]
                    e = ((lo_f32(w0) + lo_f32(w1))
                         + (lo_f32(w2) + lo_f32(w3)))
                    o = ((hi_f32(w0) + hi_f32(w1))
                         + (hi_f32(w2) + hi_f32(w3)))
                    acc_v[i0 + k + q, pl.ds(g * 32, 16)] = e
                    acc_v[i0 + k + q, pl.ds(g * 32 + 16, 16)] = o
                cur = nxt
            return 0

        lax.fori_loop(0, C // 16, blk_body, 0)
        pltpu.async_copy(acc_v, out_hbm.at[pl.ds(base, C), :], sem_out[b])

    def wait_out(b):
        pltpu.make_async_copy(acc_vs[b], out_hbm.at[pl.ds(0, C), :],
                              sem_out[b]).wait()

    def pair_body(p, _):
        for sub in range(2):
            t = p * 2 + sub
            b = sub

            @pl.when(t < nmine)
            def _():
                @pl.when(t + 1 < nmine)
                def _():
                    issue_idx(t + 1, 1 - b)
                wait_idx(b)

                @pl.when(t >= 2)
                def _():
                    wait_out(b)
                do_chunk(t, b)
        return 0

    lax.fori_loop(0, (nmine + 1) // 2, pair_body, 0)
    wait_out(0)
    wait_out(1)


def kernel(atomic_num, formal_charge, degree, chiral_tag, total_numHs,
           is_aromatic, hybridization, W_atomic_num, W_formal_charge,
           W_degree, W_chiral_tag, W_total_numHs, W_is_aromatic,
           W_hybridization):
    return _embed_sum(atomic_num, formal_charge, degree, chiral_tag,
                      total_numHs, is_aromatic, hybridization,
                      W_atomic_num, W_formal_charge, W_degree,
                      W_chiral_tag, W_total_numHs, W_is_aromatic,
                      W_hybridization)
